# baseline (speedup 1.0000x reference)
import functools

import jax
import jax.numpy as jnp
from jax import lax
from jax.experimental import pallas as pl
from jax.experimental.pallas import tpu as pltpu
from jax.experimental.pallas import tpu_sc as plsc

_CHUNK_ROWS = 400
_N_CHUNKS = 250
_NC = 2
_NS = 16
_NW = _NC * _NS


def _sc_copy(w_hbm, o_hbm, buf, sem):
    wid = lax.axis_index("s") * _NC + lax.axis_index("c")
    n_mine = (_N_CHUNKS - wid + _NW - 1) // _NW

    @pl.loop(0, n_mine)
    def _(i):
        base = (wid + i * _NW) * _CHUNK_ROWS
        pltpu.async_copy(w_hbm.at[pl.ds(base, _CHUNK_ROWS), :], buf, sem).wait()
        pltpu.sync_copy(buf, o_hbm.at[pl.ds(base, _CHUNK_ROWS), :])


def kernel(weight, edge_index):
    n, d = weight.shape
    mesh = plsc.VectorSubcoreMesh(core_axis_name="c", subcore_axis_name="s")
    k = functools.partial(
        pl.kernel,
        out_type=jax.ShapeDtypeStruct((n, d), weight.dtype),
        mesh=mesh,
        scratch_types=[
            pltpu.VMEM((_CHUNK_ROWS, d), jnp.float32),
            pltpu.SemaphoreType.DMA,
        ],
    )(_sc_copy)
    return k(weight)


# speedup vs baseline: 1.8938x; 1.8938x over previous
import jax
import jax.numpy as jnp
from jax.experimental import pallas as pl
from jax.experimental.pallas import tpu as pltpu

_D = 128
_SIZES = (2500, 17500, 20000, 20000, 20000, 17500, 2500)
_K = len(_SIZES)
_OFFS = tuple(sum(_SIZES[:i]) for i in range(_K))


def _copy_body(w_hbm, o_hbm, *rest):
    bufs, sin, sout = rest[:_K], rest[_K], rest[_K + 1]
    ins = [
        pltpu.make_async_copy(
            w_hbm.at[pl.ds(_OFFS[i], _SIZES[i]), :], bufs[i], sin.at[i]
        )
        for i in range(_K)
    ]
    outs = [
        pltpu.make_async_copy(
            bufs[i], o_hbm.at[pl.ds(_OFFS[i], _SIZES[i]), :], sout.at[i]
        )
        for i in range(_K)
    ]
    for c in ins:
        c.start()
    for i in range(_K):
        ins[i].wait()
        outs[i].start()
    for c in outs:
        c.wait()


def kernel(weight, edge_index):
    n, d = weight.shape
    return pl.pallas_call(
        _copy_body,
        out_shape=jax.ShapeDtypeStruct((n, d), weight.dtype),
        in_specs=[pl.BlockSpec(memory_space=pl.ANY)],
        out_specs=pl.BlockSpec(memory_space=pl.ANY),
        scratch_shapes=(
            [pltpu.VMEM((sz, _D), jnp.float32) for sz in _SIZES]
            + [pltpu.SemaphoreType.DMA((_K,)), pltpu.SemaphoreType.DMA((_K,))]
        ),
    )(weight)


# speedup vs baseline: 1.9120x; 1.0096x over previous
import jax
import jax.numpy as jnp
from jax.experimental import pallas as pl
from jax.experimental.pallas import tpu as pltpu

_D = 128
_SIZES = (1000, 4000, 20000, 25000, 25000, 20000, 4000, 1000)
_K = len(_SIZES)
_OFFS = tuple(sum(_SIZES[:i]) for i in range(_K))


def _copy_body(w_hbm, o_hbm, *rest):
    bufs, sin, sout = rest[:_K], rest[_K], rest[_K + 1]
    ins = [
        pltpu.make_async_copy(
            w_hbm.at[pl.ds(_OFFS[i], _SIZES[i]), :], bufs[i], sin.at[i]
        )
        for i in range(_K)
    ]
    outs = [
        pltpu.make_async_copy(
            bufs[i], o_hbm.at[pl.ds(_OFFS[i], _SIZES[i]), :], sout.at[i]
        )
        for i in range(_K)
    ]
    for c in ins:
        c.start()
    for i in range(_K):
        ins[i].wait()
        outs[i].start()
    for c in outs:
        c.wait()


def kernel(weight, edge_index):
    n, d = weight.shape
    return pl.pallas_call(
        _copy_body,
        out_shape=jax.ShapeDtypeStruct((n, d), weight.dtype),
        in_specs=[pl.BlockSpec(memory_space=pl.ANY)],
        out_specs=pl.BlockSpec(memory_space=pl.ANY),
        scratch_shapes=(
            [pltpu.VMEM((sz, _D), jnp.float32) for sz in _SIZES]
            + [pltpu.SemaphoreType.DMA((_K,)), pltpu.SemaphoreType.DMA((_K,))]
        ),
    )(weight)


# speedup vs baseline: 1.9194x; 1.0039x over previous
import jax
import jax.numpy as jnp
from jax.experimental import pallas as pl
from jax.experimental.pallas import tpu as pltpu

_D = 128
_SIZES = (500, 1500, 4000, 10000, 21000, 26000, 21000, 10000, 4000, 1500, 500)
_K = len(_SIZES)
_OFFS = tuple(sum(_SIZES[:i]) for i in range(_K))


def _copy_body(w_hbm, o_hbm, *rest):
    bufs, sin, sout = rest[:_K], rest[_K], rest[_K + 1]
    ins = [
        pltpu.make_async_copy(
            w_hbm.at[pl.ds(_OFFS[i], _SIZES[i]), :], bufs[i], sin.at[i]
        )
        for i in range(_K)
    ]
    outs = [
        pltpu.make_async_copy(
            bufs[i], o_hbm.at[pl.ds(_OFFS[i], _SIZES[i]), :], sout.at[i]
        )
        for i in range(_K)
    ]
    for c in ins:
        c.start()
    for i in range(_K):
        ins[i].wait()
        outs[i].start()
    for c in outs:
        c.wait()


def kernel(weight, edge_index):
    n, d = weight.shape
    return pl.pallas_call(
        _copy_body,
        out_shape=jax.ShapeDtypeStruct((n, d), weight.dtype),
        in_specs=[pl.BlockSpec(memory_space=pl.ANY)],
        out_specs=pl.BlockSpec(memory_space=pl.ANY),
        scratch_shapes=(
            [pltpu.VMEM((sz, _D), jnp.float32) for sz in _SIZES]
            + [pltpu.SemaphoreType.DMA((_K,)), pltpu.SemaphoreType.DMA((_K,))]
        ),
    )(weight)


# speedup vs baseline: 1.9200x; 1.0003x over previous
import jax
import jax.numpy as jnp
from jax.experimental import pallas as pl
from jax.experimental.pallas import tpu as pltpu

_D = 128
_SIZES = (250, 750, 2000, 5000, 12000, 30000, 30000, 12000, 5000, 2000, 750, 250)
_K = len(_SIZES)
_OFFS = tuple(sum(_SIZES[:i]) for i in range(_K))


def _copy_body(w_hbm, o_hbm, *rest):
    bufs, sin, sout = rest[:_K], rest[_K], rest[_K + 1]
    ins = [
        pltpu.make_async_copy(
            w_hbm.at[pl.ds(_OFFS[i], _SIZES[i]), :], bufs[i], sin.at[i]
        )
        for i in range(_K)
    ]
    outs = [
        pltpu.make_async_copy(
            bufs[i], o_hbm.at[pl.ds(_OFFS[i], _SIZES[i]), :], sout.at[i]
        )
        for i in range(_K)
    ]
    for c in ins:
        c.start()
    for i in range(_K):
        ins[i].wait()
        outs[i].start()
    for c in outs:
        c.wait()


def kernel(weight, edge_index):
    n, d = weight.shape
    return pl.pallas_call(
        _copy_body,
        out_shape=jax.ShapeDtypeStruct((n, d), weight.dtype),
        in_specs=[pl.BlockSpec(memory_space=pl.ANY)],
        out_specs=pl.BlockSpec(memory_space=pl.ANY),
        scratch_shapes=(
            [pltpu.VMEM((sz, _D), jnp.float32) for sz in _SIZES]
            + [pltpu.SemaphoreType.DMA((_K,)), pltpu.SemaphoreType.DMA((_K,))]
        ),
    )(weight)
